# Initial kernel scaffold; baseline (speedup 1.0000x reference)
import jax
import jax.numpy as jnp
from jax.experimental import pallas as pl


def kernel(x):
    raise NotImplementedError("write your pallas kernel here")


# speedup vs baseline: 2.0419x; 2.0419x over previous
import functools

import jax
import jax.numpy as jnp
from jax import lax
from jax.experimental import pallas as pl
from jax.experimental.pallas import tpu as pltpu
from jax.experimental.pallas import tpu_sc as plsc

B, M, N = 4, 4096, 2048

NUM_CORES = 2
NUM_SUBCORES = 16
LANES = 16
NUM_WORKERS = NUM_CORES * NUM_SUBCORES

STRIP_LANES = 128
NUM_STRIPS = B * (N // STRIP_LANES)
STRIPS_PER_WORKER = NUM_STRIPS // NUM_WORKERS
NVEC = STRIP_LANES // LANES
CHUNK = 512
NUM_CHUNKS = M // CHUNK
BLOCKS_PER_ROW = N // STRIP_LANES


def _body(x_hbm, out_hbm, buf, sem_in, sem_out):
    core = lax.axis_index("c")
    sub = lax.axis_index("s")
    wid = sub * NUM_CORES + core

    for k in range(STRIPS_PER_WORKER):
        sid = wid + NUM_WORKERS * k
        b = sid // BLOCKS_PER_ROW
        lane0 = pl.multiple_of((sid % BLOCKS_PER_ROW) * STRIP_LANES, STRIP_LANES)

        carries = tuple(jnp.zeros((LANES,), jnp.float32) for _ in range(NVEC))
        for ch in range(NUM_CHUNKS):
            r0 = ch * CHUNK
            src = x_hbm.at[b, pl.ds(r0, CHUNK), pl.ds(lane0, STRIP_LANES)]
            pltpu.async_copy(src, buf, sem_in).wait()

            def row(r, carry):
                new = []
                for j in range(NVEC):
                    v = carry[j] + buf[r, pl.ds(j * LANES, LANES)]
                    buf[r, pl.ds(j * LANES, LANES)] = v
                    new.append(v)
                return tuple(new)

            carries = lax.fori_loop(0, CHUNK, row, carries)

            dst = out_hbm.at[b, pl.ds(r0, CHUNK), pl.ds(lane0, STRIP_LANES)]
            pltpu.async_copy(buf, dst, sem_out).wait()


@jax.jit
def kernel(x):
    mesh = plsc.VectorSubcoreMesh(
        core_axis_name="c", subcore_axis_name="s"
    )
    run = functools.partial(
        pl.kernel,
        out_type=jax.ShapeDtypeStruct((B, M, N), jnp.float32),
        mesh=mesh,
        scratch_types=[
            pltpu.VMEM((CHUNK, STRIP_LANES), jnp.float32),
            pltpu.SemaphoreType.DMA,
            pltpu.SemaphoreType.DMA,
        ],
    )(_body)
    return run(x)


# speedup vs baseline: 2.7859x; 1.3643x over previous
import functools

import jax
import jax.numpy as jnp
from jax import lax
from jax.experimental import pallas as pl
from jax.experimental.pallas import tpu as pltpu
from jax.experimental.pallas import tpu_sc as plsc

B, M, N = 4, 4096, 2048

NUM_CORES = 2
NUM_SUBCORES = 16
LANES = 16
NUM_WORKERS = NUM_CORES * NUM_SUBCORES

STRIP_LANES = 128
NUM_STRIPS = B * (N // STRIP_LANES)
STRIPS_PER_WORKER = NUM_STRIPS // NUM_WORKERS
NVEC = STRIP_LANES // LANES
CHUNK = 256
NUM_CHUNKS = M // CHUNK
BLOCKS_PER_ROW = N // STRIP_LANES
TOTAL_ITERS = STRIPS_PER_WORKER * NUM_CHUNKS


def _hbm_slice(ref, it, wid):
    strip, ch = divmod(it, NUM_CHUNKS)
    sid = wid + NUM_WORKERS * strip
    b = sid // BLOCKS_PER_ROW
    lane0 = pl.multiple_of((sid % BLOCKS_PER_ROW) * STRIP_LANES, STRIP_LANES)
    return ref.at[b, pl.ds(ch * CHUNK, CHUNK), pl.ds(lane0, STRIP_LANES)]


def _body(x_hbm, out_hbm, buf0, buf1, sem_in0, sem_in1, sem_out0, sem_out1):
    core = lax.axis_index("c")
    sub = lax.axis_index("s")
    wid = sub * NUM_CORES + core

    bufs = (buf0, buf1)
    sems_in = (sem_in0, sem_in1)
    sems_out = (sem_out0, sem_out1)
    in_copies = [None, None]
    out_copies = [None, None]

    in_copies[0] = pltpu.async_copy(_hbm_slice(x_hbm, 0, wid), bufs[0], sems_in[0])

    carries = None
    for it in range(TOTAL_ITERS):
        nb = it % 2
        ot = 1 - nb
        if it + 1 < TOTAL_ITERS:
            if out_copies[ot] is not None:
                out_copies[ot].wait()
                out_copies[ot] = None
            in_copies[ot] = pltpu.async_copy(
                _hbm_slice(x_hbm, it + 1, wid), bufs[ot], sems_in[ot]
            )

        if it % NUM_CHUNKS == 0:
            carries = tuple(jnp.zeros((LANES,), jnp.float32) for _ in range(NVEC))

        in_copies[nb].wait()
        in_copies[nb] = None
        buf = bufs[nb]

        def row(r, carry):
            new = []
            for j in range(NVEC):
                v = carry[j] + buf[r, pl.ds(j * LANES, LANES)]
                buf[r, pl.ds(j * LANES, LANES)] = v
                new.append(v)
            return tuple(new)

        carries = lax.fori_loop(0, CHUNK, row, carries)

        out_copies[nb] = pltpu.async_copy(
            buf, _hbm_slice(out_hbm, it, wid), sems_out[nb]
        )

    for nb in range(2):
        if out_copies[nb] is not None:
            out_copies[nb].wait()


@jax.jit
def kernel(x):
    mesh = plsc.VectorSubcoreMesh(
        core_axis_name="c", subcore_axis_name="s"
    )
    run = functools.partial(
        pl.kernel,
        out_type=jax.ShapeDtypeStruct((B, M, N), jnp.float32),
        mesh=mesh,
        scratch_types=[
            pltpu.VMEM((CHUNK, STRIP_LANES), jnp.float32),
            pltpu.VMEM((CHUNK, STRIP_LANES), jnp.float32),
            pltpu.SemaphoreType.DMA,
            pltpu.SemaphoreType.DMA,
            pltpu.SemaphoreType.DMA,
            pltpu.SemaphoreType.DMA,
        ],
    )(_body)
    return run(x)


# speedup vs baseline: 2.7886x; 1.0010x over previous
import functools

import jax
import jax.numpy as jnp
from jax import lax
from jax.experimental import pallas as pl
from jax.experimental.pallas import tpu as pltpu
from jax.experimental.pallas import tpu_sc as plsc

B, M, N = 4, 4096, 2048

NUM_CORES = 2
NUM_SUBCORES = 16
LANES = 16
NUM_WORKERS = NUM_CORES * NUM_SUBCORES

STRIP_LANES = 128
NUM_STRIPS = B * (N // STRIP_LANES)
STRIPS_PER_WORKER = NUM_STRIPS // NUM_WORKERS
NVEC = STRIP_LANES // LANES
CHUNK = 256
NUM_CHUNKS = M // CHUNK
BLOCKS_PER_ROW = N // STRIP_LANES
TOTAL_ITERS = STRIPS_PER_WORKER * NUM_CHUNKS


def _hbm_slice(ref, it, wid):
    strip, ch = divmod(it, NUM_CHUNKS)
    sid = wid + NUM_WORKERS * strip
    b = sid // BLOCKS_PER_ROW
    lane0 = pl.multiple_of((sid % BLOCKS_PER_ROW) * STRIP_LANES, STRIP_LANES)
    return ref.at[b, pl.ds(ch * CHUNK, CHUNK), pl.ds(lane0, STRIP_LANES)]


NBUF = 3


def _body(x_hbm, out_hbm, *refs):
    bufs = refs[:NBUF]
    sems_in = refs[NBUF : 2 * NBUF]
    sems_out = refs[2 * NBUF : 3 * NBUF]
    core = lax.axis_index("c")
    sub = lax.axis_index("s")
    wid = sub * NUM_CORES + core

    in_copies = [None] * NBUF
    out_copies = [None] * NBUF

    for it in range(min(NBUF, TOTAL_ITERS)):
        in_copies[it] = pltpu.async_copy(
            _hbm_slice(x_hbm, it, wid), bufs[it], sems_in[it]
        )

    carries = None
    for it in range(TOTAL_ITERS):
        nb = it % NBUF

        nxt = it + NBUF - 1
        if nxt < TOTAL_ITERS and in_copies[nxt % NBUF] is None:
            s = nxt % NBUF
            if out_copies[s] is not None:
                out_copies[s].wait()
                out_copies[s] = None
            in_copies[s] = pltpu.async_copy(
                _hbm_slice(x_hbm, nxt, wid), bufs[s], sems_in[s]
            )

        if it % NUM_CHUNKS == 0:
            carries = tuple(jnp.zeros((LANES,), jnp.float32) for _ in range(NVEC))

        in_copies[nb].wait()
        in_copies[nb] = None
        buf = bufs[nb]

        def row(r, carry):
            new = []
            for j in range(NVEC):
                v = carry[j] + buf[r, pl.ds(j * LANES, LANES)]
                buf[r, pl.ds(j * LANES, LANES)] = v
                new.append(v)
            return tuple(new)

        carries = lax.fori_loop(0, CHUNK, row, carries)

        out_copies[nb] = pltpu.async_copy(
            buf, _hbm_slice(out_hbm, it, wid), sems_out[nb]
        )

    for nb in range(NBUF):
        if out_copies[nb] is not None:
            out_copies[nb].wait()


@jax.jit
def kernel(x):
    mesh = plsc.VectorSubcoreMesh(
        core_axis_name="c", subcore_axis_name="s"
    )
    run = functools.partial(
        pl.kernel,
        out_type=jax.ShapeDtypeStruct((B, M, N), jnp.float32),
        mesh=mesh,
        scratch_types=(
            [pltpu.VMEM((CHUNK, STRIP_LANES), jnp.float32)] * NBUF
            + [pltpu.SemaphoreType.DMA] * (2 * NBUF)
        ),
    )(_body)
    return run(x)
